# baseline (speedup 1.0000x reference)
import jax
import jax.numpy as jnp
from jax.experimental import pallas as pl
from jax.experimental.pallas import tpu as pltpu

_ROWS_PER_TILE = 8
_TILES_PER_BLOCK = 10


def _scale_body(ei_ref, aw_ref, x_ref, o_ref, w_ref):
    m = ei_ref.shape[0]
    num_l = aw_ref.shape[0]

    @pl.when(pl.program_id(0) == 0)
    def _():
        ei = ei_ref[...]
        onehot = (ei == jax.lax.broadcasted_iota(jnp.int32, (m, num_l), 1))
        w49 = jax.lax.dot_general(
            onehot.astype(jnp.float32), aw_ref[...],
            (((1,), (0,)), ((), ())),
            preferred_element_type=jnp.float32)
        for k in range(_ROWS_PER_TILE):
            w_ref[pl.ds(k * m, m), :] = w49

    for j in range(_TILES_PER_BLOCK):
        sl = pl.ds(j * m * _ROWS_PER_TILE, m * _ROWS_PER_TILE)
        o_ref[sl, :] = x_ref[sl, :] * w_ref[...]


def kernel(node_input, affine_weight, expand_index):
    n, m, c = node_input.shape
    aw = affine_weight.reshape(affine_weight.shape[-2], c)
    ei = expand_index.astype(jnp.int32).reshape(m, 1)
    x = node_input.reshape(n * m, c)

    block_rows = m * _ROWS_PER_TILE * _TILES_PER_BLOCK
    grid = (n * m // block_rows,)
    out = pl.pallas_call(
        _scale_body,
        grid=grid,
        in_specs=[
            pl.BlockSpec((m, 1), lambda i: (0, 0)),
            pl.BlockSpec(aw.shape, lambda i: (0, 0)),
            pl.BlockSpec((block_rows, c), lambda i: (i, 0)),
        ],
        out_specs=pl.BlockSpec((block_rows, c), lambda i: (i, 0)),
        out_shape=jax.ShapeDtypeStruct((n * m, c), jnp.float32),
        scratch_shapes=[pltpu.VMEM((m * _ROWS_PER_TILE, c), jnp.float32)],
    )(ei, aw, x)
    return out.reshape(n, m, c)


# speedup vs baseline: 2.3855x; 2.3855x over previous
import jax
import jax.numpy as jnp
from jax.experimental import pallas as pl
from jax.experimental.pallas import tpu as pltpu

_BLOCK_NODES = 100


def _scale_body(ei_ref, aw_ref, x_ref, o_ref, w_ref):
    m = ei_ref.shape[0]
    num_l = aw_ref.shape[0]

    @pl.when(pl.program_id(0) == 0)
    def _():
        ei = ei_ref[...]
        onehot = (ei == jax.lax.broadcasted_iota(jnp.int32, (m, num_l), 1))
        w_ref[...] = jax.lax.dot_general(
            onehot.astype(jnp.float32), aw_ref[...],
            (((1,), (0,)), ((), ())),
            preferred_element_type=jnp.float32)

    o_ref[...] = x_ref[...] * w_ref[...][None]


def kernel(node_input, affine_weight, expand_index):
    n, m, c = node_input.shape
    aw = affine_weight.reshape(affine_weight.shape[-2], c)
    ei = expand_index.astype(jnp.int32).reshape(m, 1)

    bn = _BLOCK_NODES
    grid = (n // bn,)
    return pl.pallas_call(
        _scale_body,
        grid=grid,
        in_specs=[
            pl.BlockSpec((m, 1), lambda i: (0, 0)),
            pl.BlockSpec(aw.shape, lambda i: (0, 0)),
            pl.BlockSpec((bn, m, c), lambda i: (i, 0, 0)),
        ],
        out_specs=pl.BlockSpec((bn, m, c), lambda i: (i, 0, 0)),
        out_shape=jax.ShapeDtypeStruct((n, m, c), jnp.float32),
        scratch_shapes=[pltpu.VMEM((m, c), jnp.float32)],
    )(ei, aw, node_input)


# speedup vs baseline: 2.4055x; 1.0084x over previous
import jax
import jax.numpy as jnp
from jax.experimental import pallas as pl
from jax.experimental.pallas import tpu as pltpu

_BLOCK_NODES = 250


def _scale_body(ei_ref, aw_ref, x_ref, o_ref, w_ref):
    m = ei_ref.shape[0]
    num_l = aw_ref.shape[0]

    @pl.when(pl.program_id(0) == 0)
    def _():
        ei = ei_ref[...]
        onehot = (ei == jax.lax.broadcasted_iota(jnp.int32, (m, num_l), 1))
        w_ref[...] = jax.lax.dot_general(
            onehot.astype(jnp.float32), aw_ref[...],
            (((1,), (0,)), ((), ())),
            preferred_element_type=jnp.float32)

    o_ref[...] = x_ref[...] * w_ref[...][None]


def kernel(node_input, affine_weight, expand_index):
    n, m, c = node_input.shape
    aw = affine_weight.reshape(affine_weight.shape[-2], c)
    ei = expand_index.astype(jnp.int32).reshape(m, 1)

    bn = _BLOCK_NODES
    grid = (n // bn,)
    return pl.pallas_call(
        _scale_body,
        grid=grid,
        in_specs=[
            pl.BlockSpec((m, 1), lambda i: (0, 0)),
            pl.BlockSpec(aw.shape, lambda i: (0, 0)),
            pl.BlockSpec((bn, m, c), lambda i: (i, 0, 0)),
        ],
        out_specs=pl.BlockSpec((bn, m, c), lambda i: (i, 0, 0)),
        out_shape=jax.ShapeDtypeStruct((n, m, c), jnp.float32),
        scratch_shapes=[pltpu.VMEM((m, c), jnp.float32)],
    )(ei, aw, node_input)


# speedup vs baseline: 2.5076x; 1.0424x over previous
import jax
import jax.numpy as jnp
from jax.experimental import pallas as pl
from jax.experimental.pallas import tpu as pltpu

_BLOCK_NODES = 100
_NBUF = 4


def _scale_body(ei_ref, aw_ref, x_hbm, o_hbm, ibuf, obuf, w_ref, isem, osem):
    m, c = w_ref.shape
    num_l = aw_ref.shape[0]
    n = x_hbm.shape[0]
    bn = _BLOCK_NODES
    chunks = n // bn

    ei = ei_ref[...]
    onehot = (ei == jax.lax.broadcasted_iota(jnp.int32, (m, num_l), 1))
    w_ref[...] = jax.lax.dot_general(
        onehot.astype(jnp.float32), aw_ref[...],
        (((1,), (0,)), ((), ())),
        preferred_element_type=jnp.float32)

    def in_copy(i, b):
        return pltpu.make_async_copy(
            x_hbm.at[pl.ds(i * bn, bn)], ibuf.at[b], isem.at[b])

    def out_copy(i, b):
        return pltpu.make_async_copy(
            obuf.at[b], o_hbm.at[pl.ds(i * bn, bn)], osem.at[b])

    for b in range(min(_NBUF, chunks)):
        in_copy(b, b).start()

    for i in range(chunks):
        b = i % _NBUF
        in_copy(i, b).wait()
        if i >= _NBUF:
            out_copy(i - _NBUF, b).wait()
        obuf[b] = ibuf[b] * w_ref[...][None]
        if i + _NBUF < chunks:
            in_copy(i + _NBUF, b).start()
        out_copy(i, b).start()

    for i in range(max(chunks - _NBUF, 0), chunks):
        out_copy(i, i % _NBUF).wait()


def kernel(node_input, affine_weight, expand_index):
    n, m, c = node_input.shape
    aw = affine_weight.reshape(affine_weight.shape[-2], c)
    ei = expand_index.astype(jnp.int32).reshape(m, 1)
    bn = _BLOCK_NODES

    return pl.pallas_call(
        _scale_body,
        in_specs=[
            pl.BlockSpec(memory_space=pltpu.MemorySpace.VMEM),
            pl.BlockSpec(memory_space=pltpu.MemorySpace.VMEM),
            pl.BlockSpec(memory_space=pltpu.MemorySpace.HBM),
        ],
        out_specs=pl.BlockSpec(memory_space=pltpu.MemorySpace.HBM),
        out_shape=jax.ShapeDtypeStruct((n, m, c), jnp.float32),
        scratch_shapes=[
            pltpu.VMEM((_NBUF, bn, m, c), jnp.float32),
            pltpu.VMEM((_NBUF, bn, m, c), jnp.float32),
            pltpu.VMEM((m, c), jnp.float32),
            pltpu.SemaphoreType.DMA((_NBUF,)),
            pltpu.SemaphoreType.DMA((_NBUF,)),
        ],
    )(ei, aw, node_input)


# speedup vs baseline: 2.5093x; 1.0007x over previous
import jax
import jax.numpy as jnp
from jax.experimental import pallas as pl
from jax.experimental.pallas import tpu as pltpu

_BLOCK_NODES = 50
_NBUF = 8


def _scale_body(ei_ref, aw_ref, x_hbm, o_hbm, ibuf, obuf, w_ref, isem, osem):
    m, c = w_ref.shape
    num_l = aw_ref.shape[0]
    n = x_hbm.shape[0]
    bn = _BLOCK_NODES
    chunks = n // bn

    ei = ei_ref[...]
    onehot = (ei == jax.lax.broadcasted_iota(jnp.int32, (m, num_l), 1))
    w_ref[...] = jax.lax.dot_general(
        onehot.astype(jnp.float32), aw_ref[...],
        (((1,), (0,)), ((), ())),
        preferred_element_type=jnp.float32)

    def in_copy(i, b):
        return pltpu.make_async_copy(
            x_hbm.at[pl.ds(i * bn, bn)], ibuf.at[b], isem.at[b])

    def out_copy(i, b):
        return pltpu.make_async_copy(
            obuf.at[b], o_hbm.at[pl.ds(i * bn, bn)], osem.at[b])

    for b in range(min(_NBUF, chunks)):
        in_copy(b, b).start()

    for i in range(chunks):
        b = i % _NBUF
        in_copy(i, b).wait()
        if i >= _NBUF:
            out_copy(i - _NBUF, b).wait()
        obuf[b] = ibuf[b] * w_ref[...][None]
        if i + _NBUF < chunks:
            in_copy(i + _NBUF, b).start()
        out_copy(i, b).start()

    for i in range(max(chunks - _NBUF, 0), chunks):
        out_copy(i, i % _NBUF).wait()


def kernel(node_input, affine_weight, expand_index):
    n, m, c = node_input.shape
    aw = affine_weight.reshape(affine_weight.shape[-2], c)
    ei = expand_index.astype(jnp.int32).reshape(m, 1)
    bn = _BLOCK_NODES

    return pl.pallas_call(
        _scale_body,
        in_specs=[
            pl.BlockSpec(memory_space=pltpu.MemorySpace.VMEM),
            pl.BlockSpec(memory_space=pltpu.MemorySpace.VMEM),
            pl.BlockSpec(memory_space=pltpu.MemorySpace.HBM),
        ],
        out_specs=pl.BlockSpec(memory_space=pltpu.MemorySpace.HBM),
        out_shape=jax.ShapeDtypeStruct((n, m, c), jnp.float32),
        scratch_shapes=[
            pltpu.VMEM((_NBUF, bn, m, c), jnp.float32),
            pltpu.VMEM((_NBUF, bn, m, c), jnp.float32),
            pltpu.VMEM((m, c), jnp.float32),
            pltpu.SemaphoreType.DMA((_NBUF,)),
            pltpu.SemaphoreType.DMA((_NBUF,)),
        ],
    )(ei, aw, node_input)
